# baseline (speedup 1.0000x reference)
import jax
import jax.numpy as jnp
from jax import lax
from jax.experimental import pallas as pl
from jax.experimental.pallas import tpu as pltpu

_ANCH_W = (10.0, 16.0, 33.0)
_ANCH_H = (13.0, 30.0, 23.0)
_GS = 52
_G = _GS * _GS
_NA = 3
_NF = 85
_STRIDE = 8.0


def _body(x_ref, o_ref):
    v = x_ref[0]

    shp = (_NA * _G, _NF)
    k = lax.broadcasted_iota(jnp.int32, shp, 1)
    n = lax.broadcasted_iota(jnp.int32, shp, 0)
    g = n % _G
    gx = (g % _GS).astype(jnp.float32)
    gy = (g // _GS).astype(jnp.float32)
    a0 = n < _G
    a1 = (n >= _G) & (n < 2 * _G)

    aw = jnp.where(a0, _ANCH_W[0], jnp.where(a1, _ANCH_W[1], _ANCH_W[2]))
    ah = jnp.where(a0, _ANCH_H[0], jnp.where(a1, _ANCH_H[1], _ANCH_H[2]))

    sig = jax.nn.sigmoid(v)
    ex = jnp.exp(v)
    is_wh = (k == 2) | (k == 3)
    base = jnp.where(is_wh, ex, sig)
    scale = jnp.where(k < 2, _STRIDE,
                      jnp.where(k == 2, aw, jnp.where(k == 3, ah, 1.0)))
    grid_term = jnp.where(k == 0, gx, jnp.where(k == 1, gy, 0.0))
    o_ref[0] = base * scale + grid_term * _STRIDE


def kernel(inputs):
    b = inputs.shape[0]
    x = jnp.transpose(inputs.reshape(b, _NA, _NF, _GS, _GS),
                      (0, 1, 3, 4, 2)).reshape(b, _NA * _G, _NF)
    out = pl.pallas_call(
        _body,
        grid=(b,),
        in_specs=[pl.BlockSpec((1, _NA * _G, _NF), lambda i: (i, 0, 0))],
        out_specs=pl.BlockSpec((1, _NA * _G, _NF), lambda i: (i, 0, 0)),
        out_shape=jax.ShapeDtypeStruct((b, _NA * _G, _NF), jnp.float32),
        compiler_params=pltpu.CompilerParams(
            dimension_semantics=("parallel",)),
    )(x)
    return (out, 0, 0)


# speedup vs baseline: 1.4237x; 1.4237x over previous
import jax
import jax.numpy as jnp
from jax import lax
from jax.experimental import pallas as pl

_ANCH_W = (10.0, 16.0, 33.0)
_ANCH_H = (13.0, 30.0, 23.0)
_GS = 52
_G = _GS * _GS
_NA = 3
_NF = 85
_STRIDE = 8.0


def _body(x_ref, o_ref):
    a = pl.program_id(1)
    v = x_ref[0, 0].astype(jnp.float32)

    aw = jnp.where(a == 0, _ANCH_W[0], jnp.where(a == 1, _ANCH_W[1], _ANCH_W[2]))
    ah = jnp.where(a == 0, _ANCH_H[0], jnp.where(a == 1, _ANCH_H[1], _ANCH_H[2]))

    g = lax.broadcasted_iota(jnp.int32, (2, _G), 1)
    r = lax.broadcasted_iota(jnp.int32, (2, _G), 0)
    grid_off = jnp.where(r == 0, g % _GS, g // _GS).astype(jnp.float32)

    xy = (jax.nn.sigmoid(v[0:2, :]) + grid_off) * _STRIDE
    wh = jnp.exp(v[2:4, :]) * jnp.where(
        lax.broadcasted_iota(jnp.int32, (2, _G), 0) == 0, aw, ah)
    rest = jax.nn.sigmoid(v[4:, :])

    full = jnp.concatenate(
        [xy, wh, rest, jnp.zeros((128 - _NF, _G), jnp.float32)], axis=0)
    o_ref[0] = full.T


def kernel(inputs):
    b = inputs.shape[0]
    x = inputs.astype(jnp.bfloat16).reshape(b, _NA, _NF, _G)
    out = pl.pallas_call(
        _body,
        grid=(b, _NA),
        in_specs=[pl.BlockSpec((1, 1, _NF, _G), lambda i, j: (i, j, 0, 0))],
        out_specs=pl.BlockSpec((1, _G, 128), lambda i, j: (i, j, 0)),
        out_shape=jax.ShapeDtypeStruct((b, _NA * _G, 128), jnp.float32),
    )(x)
    return (out[:, :, :_NF], 0, 0)
